# Initial kernel scaffold; baseline (speedup 1.0000x reference)
import jax
import jax.numpy as jnp
from jax.experimental import pallas as pl


def kernel(x, edge_index, W1, b1, W2, b2, W3, b3):
    raise NotImplementedError("write your pallas kernel here")


# speedup vs baseline: 4.0283x; 4.0283x over previous
import functools

import jax
import jax.numpy as jnp
from jax import lax
from jax.experimental import pallas as pl
from jax.experimental.pallas import tpu as pltpu
from jax.experimental.pallas import tpu_sc as plsc

_N = 10000
_E = 320000
_D = 128

_NC = 2
_NS = 16
_NW = _NC * _NS
_EW = _E // _NW
_K = 80
_NCH = _EW // _K
_NP = 10240
_RPS = _NP // _NS
_DG = 16


def _sc_mesh():
  return plsc.VectorSubcoreMesh(core_axis_name="c", subcore_axis_name="s")


@functools.partial(
    pl.kernel,
    out_type=jax.ShapeDtypeStruct((_NC, _NP, _D), jnp.float32),
    mesh=_sc_mesh(),
    scratch_types=[
        pltpu.VMEM((_K,), jnp.int32),
        pltpu.VMEM((_K,), jnp.int32),
        pltpu.VMEM((_K, _D), jnp.float32),
        pltpu.VMEM((_K, _D), jnp.float32),
        pltpu.VMEM_SHARED((_NP, _D), jnp.float32),
    ],
)
def _sc_degrees(src_hbm, dst_hbm, onesa_hbm, onesb_hbm, z_hbm, out_hbm,
                idx_s, idx_d, va, vb, acc):
  cid = lax.axis_index("c")
  sid = lax.axis_index("s")
  wid = sid * _NC + cid
  row0 = sid * _RPS
  pltpu.sync_copy(z_hbm, acc.at[pl.ds(row0, _RPS)])
  pltpu.sync_copy(onesa_hbm, va)
  pltpu.sync_copy(onesb_hbm, vb)
  plsc.subcore_barrier()

  def body(i, carry):
    base = pl.multiple_of(wid * _EW + i * _K, 8)
    pltpu.sync_copy(src_hbm.at[pl.ds(base, _K)], idx_s)
    pltpu.sync_copy(dst_hbm.at[pl.ds(base, _K)], idx_d)
    pltpu.sync_copy(va, acc.at[idx_s], add=True)
    pltpu.sync_copy(vb, acc.at[idx_d], add=True)
    return carry

  lax.fori_loop(0, _NCH, body, 0)
  plsc.subcore_barrier()
  pltpu.sync_copy(acc.at[pl.ds(row0, _RPS)], out_hbm.at[cid, pl.ds(row0, _RPS)])


@functools.partial(
    pl.kernel,
    out_type=jax.ShapeDtypeStruct((_NC, _NP, _D), jnp.float32),
    mesh=_sc_mesh(),
    scratch_types=[
        pltpu.VMEM((_K,), jnp.int32),
        pltpu.VMEM((_K,), jnp.int32),
        pltpu.VMEM((_K, _D), jnp.float32),
        pltpu.VMEM_SHARED((_NP, _D), jnp.float32),
        pltpu.SemaphoreType.DMA,
    ],
)
def _sc_aggregate(g_hbm, src_hbm, dst_hbm, z_hbm, out_hbm,
                  idx_s, idx_d, rows, acc, sem):
  cid = lax.axis_index("c")
  sid = lax.axis_index("s")
  wid = sid * _NC + cid
  row0 = sid * _RPS
  pltpu.sync_copy(z_hbm, acc.at[pl.ds(row0, _RPS)])
  plsc.subcore_barrier()

  def body(i, carry):
    base = pl.multiple_of(wid * _EW + i * _K, 8)
    pltpu.sync_copy(src_hbm.at[pl.ds(base, _K)], idx_s)
    pltpu.sync_copy(dst_hbm.at[pl.ds(base, _K)], idx_d)
    pltpu.async_copy(g_hbm.at[idx_s], rows, sem).wait()
    pltpu.sync_copy(rows, acc.at[idx_d], add=True)
    return carry

  lax.fori_loop(0, _NCH, body, 0)
  plsc.subcore_barrier()
  pltpu.sync_copy(acc.at[pl.ds(row0, _RPS)], out_hbm.at[cid, pl.ds(row0, _RPS)])


_R = 1280
_GB = _NP // _R


def _prep_body(x_ref, deg_ref, g_ref, ns_ref, nd_ref):
  d = deg_ref[0] + deg_ref[1]
  ns = lax.rsqrt(jnp.maximum(d[:, 0:1], 1.0))
  nd = lax.rsqrt(jnp.maximum(d[:, 64:65], 1.0))
  ns_ref[...] = jnp.broadcast_to(ns, (_R, _DG))
  nd_ref[...] = jnp.broadcast_to(nd, (_R, _DG))
  g_ref[...] = x_ref[...] * ns


def _tc_prep(x, deg):
  return pl.pallas_call(
      _prep_body,
      grid=(_GB,),
      in_specs=[
          pl.BlockSpec((_R, _D), lambda i: (i, 0)),
          pl.BlockSpec((_NC, _R, _D), lambda i: (0, i, 0)),
      ],
      out_specs=[
          pl.BlockSpec((_R, _D), lambda i: (i, 0)),
          pl.BlockSpec((_R, _DG), lambda i: (i, 0)),
          pl.BlockSpec((_R, _DG), lambda i: (i, 0)),
      ],
      out_shape=[
          jax.ShapeDtypeStruct((_N, _D), jnp.float32),
          jax.ShapeDtypeStruct((_N, _DG), jnp.float32),
          jax.ShapeDtypeStruct((_N, _DG), jnp.float32),
      ],
  )(x, deg)


def _layer_body(acc_ref, nd_ref, ns_ref, w_ref, b_ref, o_ref, *, relu, scale_next):
  t = (acc_ref[0] + acc_ref[1]) * nd_ref[...][:, :1]
  y = jnp.dot(t, w_ref[...], preferred_element_type=jnp.float32) + b_ref[...]
  if relu:
    y = jnp.maximum(y, 0.0)
  if scale_next:
    y = y * ns_ref[...][:, :1]
  o_ref[...] = y


def _tc_layer(acc, nd, ns, w, b, relu, scale_next):
  body = functools.partial(_layer_body, relu=relu, scale_next=scale_next)
  return pl.pallas_call(
      body,
      grid=(_GB,),
      in_specs=[
          pl.BlockSpec((_NC, _R, _D), lambda i: (0, i, 0)),
          pl.BlockSpec((_R, _DG), lambda i: (i, 0)),
          pl.BlockSpec((_R, _DG), lambda i: (i, 0)),
          pl.BlockSpec((_D, _D), lambda i: (0, 0)),
          pl.BlockSpec((1, _D), lambda i: (0, 0)),
      ],
      out_specs=pl.BlockSpec((_R, _D), lambda i: (i, 0)),
      out_shape=jax.ShapeDtypeStruct((_N, _D), jnp.float32),
  )(acc, nd, ns, w, b)


def kernel(x, edge_index, W1, b1, W2, b2, W3, b3):
  src = edge_index[0].astype(jnp.int32)
  dst = edge_index[1].astype(jnp.int32)
  zd = jnp.zeros((_RPS, _D), jnp.float32)
  onesa = jnp.zeros((_K, _D), jnp.float32).at[:, 0].set(1.0)
  onesb = jnp.zeros((_K, _D), jnp.float32).at[:, 64].set(1.0)

  deg = _sc_degrees(src, dst, onesa, onesb, zd)
  g, ns, nd = _tc_prep(x, deg)

  acc = _sc_aggregate(g, src, dst, zd)
  g = _tc_layer(acc, nd, ns, W1, b1.reshape(1, _D), relu=True, scale_next=True)

  acc = _sc_aggregate(g, src, dst, zd)
  g = _tc_layer(acc, nd, ns, W2, b2.reshape(1, _D), relu=True, scale_next=True)

  acc = _sc_aggregate(g, src, dst, zd)
  return _tc_layer(acc, nd, ns, W3, b3.reshape(1, _D), relu=False, scale_next=False)


# speedup vs baseline: 5.7966x; 1.4390x over previous
import functools

import jax
import jax.numpy as jnp
from jax import lax
from jax.experimental import pallas as pl
from jax.experimental.pallas import tpu as pltpu
from jax.experimental.pallas import tpu_sc as plsc

_N = 10000
_E = 320000
_D = 128

_NC = 2
_NS = 16
_NW = _NC * _NS
_EW = _E // _NW
_K = 80
_NCH = _EW // _K
_NP = 10240
_RPS = _NP // _NS
_DG = 16


def _sc_mesh():
  return plsc.VectorSubcoreMesh(core_axis_name="c", subcore_axis_name="s")


@functools.partial(
    pl.kernel,
    out_type=jax.ShapeDtypeStruct((_NC, _NP, _D), jnp.float32),
    mesh=_sc_mesh(),
    scratch_types=[
        pltpu.VMEM((_K,), jnp.int32),
        pltpu.VMEM((_K,), jnp.int32),
        pltpu.VMEM((_K, _D), jnp.float32),
        pltpu.VMEM((_K, _D), jnp.float32),
        pltpu.VMEM_SHARED((_NP, _D), jnp.float32),
    ],
)
def _sc_degrees(src_hbm, dst_hbm, onesa_hbm, onesb_hbm, z_hbm, out_hbm,
                idx_s, idx_d, va, vb, acc):
  cid = lax.axis_index("c")
  sid = lax.axis_index("s")
  wid = sid * _NC + cid
  row0 = sid * _RPS
  pltpu.sync_copy(z_hbm, acc.at[pl.ds(row0, _RPS)])
  pltpu.sync_copy(onesa_hbm, va)
  pltpu.sync_copy(onesb_hbm, vb)
  plsc.subcore_barrier()

  def body(i, carry):
    base = pl.multiple_of(wid * _EW + i * _K, 8)
    pltpu.sync_copy(src_hbm.at[pl.ds(base, _K)], idx_s)
    pltpu.sync_copy(dst_hbm.at[pl.ds(base, _K)], idx_d)
    pltpu.sync_copy(va, acc.at[idx_s], add=True)
    pltpu.sync_copy(vb, acc.at[idx_d], add=True)
    return carry

  lax.fori_loop(0, _NCH, body, 0)
  plsc.subcore_barrier()
  pltpu.sync_copy(acc.at[pl.ds(row0, _RPS)], out_hbm.at[cid, pl.ds(row0, _RPS)])


@functools.partial(
    pl.kernel,
    out_type=jax.ShapeDtypeStruct((_NC, _NP, _D), jnp.float32),
    mesh=_sc_mesh(),
    scratch_types=[
        pltpu.VMEM((_K,), jnp.int32),
        pltpu.VMEM((_K,), jnp.int32),
        pltpu.VMEM((_K,), jnp.int32),
        pltpu.VMEM((_K,), jnp.int32),
        pltpu.VMEM((_K, _D), jnp.float32),
        pltpu.VMEM((_K, _D), jnp.float32),
        pltpu.VMEM_SHARED((_NP, _D), jnp.float32),
        pltpu.SemaphoreType.DMA,
        pltpu.SemaphoreType.DMA,
    ],
)
def _sc_aggregate(g_hbm, src_hbm, dst_hbm, z_hbm, out_hbm,
                  idx_s0, idx_s1, idx_d0, idx_d1, rows0, rows1, acc,
                  sem0, sem1):
  cid = lax.axis_index("c")
  sid = lax.axis_index("s")
  wid = sid * _NC + cid
  row0 = sid * _RPS
  pltpu.sync_copy(z_hbm, acc.at[pl.ds(row0, _RPS)])
  plsc.subcore_barrier()

  bufs = ((idx_s0, idx_d0, rows0, sem0), (idx_s1, idx_d1, rows1, sem1))
  e0 = wid * _EW

  def prefetch(ch, b):
    s, d, r, sm = bufs[b]
    base = pl.multiple_of(e0 + ch * _K, 8)
    pltpu.sync_copy(src_hbm.at[pl.ds(base, _K)], s)
    pltpu.sync_copy(dst_hbm.at[pl.ds(base, _K)], d)
    pltpu.async_copy(g_hbm.at[s], r, sm)

  prefetch(0, 0)
  prefetch(1, 1)

  def pair(p, carry):
    for b in range(2):
      ch = p * 2 + b
      s, d, r, sm = bufs[b]

      @pl.when(ch < _NCH)
      def _():
        pltpu.make_async_copy(g_hbm.at[s], r, sm).wait()
        pltpu.sync_copy(r, acc.at[d], add=True)

        @pl.when(ch + 2 < _NCH)
        def _():
          prefetch(ch + 2, b)

    return carry

  lax.fori_loop(0, (_NCH + 1) // 2, pair, 0)
  plsc.subcore_barrier()
  pltpu.sync_copy(acc.at[pl.ds(row0, _RPS)], out_hbm.at[cid, pl.ds(row0, _RPS)])


_R = 1280
_GB = _NP // _R


def _prep_body(x_ref, deg_ref, g_ref, ns_ref, nd_ref):
  d = deg_ref[0] + deg_ref[1]
  ns = lax.rsqrt(jnp.maximum(d[:, 0:1], 1.0))
  nd = lax.rsqrt(jnp.maximum(d[:, 64:65], 1.0))
  ns_ref[...] = jnp.broadcast_to(ns, (_R, _DG))
  nd_ref[...] = jnp.broadcast_to(nd, (_R, _DG))
  g_ref[...] = x_ref[...] * ns


def _tc_prep(x, deg):
  return pl.pallas_call(
      _prep_body,
      grid=(_GB,),
      in_specs=[
          pl.BlockSpec((_R, _D), lambda i: (i, 0)),
          pl.BlockSpec((_NC, _R, _D), lambda i: (0, i, 0)),
      ],
      out_specs=[
          pl.BlockSpec((_R, _D), lambda i: (i, 0)),
          pl.BlockSpec((_R, _DG), lambda i: (i, 0)),
          pl.BlockSpec((_R, _DG), lambda i: (i, 0)),
      ],
      out_shape=[
          jax.ShapeDtypeStruct((_N, _D), jnp.float32),
          jax.ShapeDtypeStruct((_N, _DG), jnp.float32),
          jax.ShapeDtypeStruct((_N, _DG), jnp.float32),
      ],
  )(x, deg)


def _layer_body(acc_ref, nd_ref, ns_ref, w_ref, b_ref, o_ref, *, relu, scale_next):
  t = (acc_ref[0] + acc_ref[1]) * nd_ref[...][:, :1]
  y = jnp.dot(t, w_ref[...], preferred_element_type=jnp.float32) + b_ref[...]
  if relu:
    y = jnp.maximum(y, 0.0)
  if scale_next:
    y = y * ns_ref[...][:, :1]
  o_ref[...] = y


def _tc_layer(acc, nd, ns, w, b, relu, scale_next):
  body = functools.partial(_layer_body, relu=relu, scale_next=scale_next)
  return pl.pallas_call(
      body,
      grid=(_GB,),
      in_specs=[
          pl.BlockSpec((_NC, _R, _D), lambda i: (0, i, 0)),
          pl.BlockSpec((_R, _DG), lambda i: (i, 0)),
          pl.BlockSpec((_R, _DG), lambda i: (i, 0)),
          pl.BlockSpec((_D, _D), lambda i: (0, 0)),
          pl.BlockSpec((1, _D), lambda i: (0, 0)),
      ],
      out_specs=pl.BlockSpec((_R, _D), lambda i: (i, 0)),
      out_shape=jax.ShapeDtypeStruct((_N, _D), jnp.float32),
  )(acc, nd, ns, w, b)


def kernel(x, edge_index, W1, b1, W2, b2, W3, b3):
  src = edge_index[0].astype(jnp.int32)
  dst = edge_index[1].astype(jnp.int32)
  zd = jnp.zeros((_RPS, _D), jnp.float32)
  onesa = jnp.zeros((_K, _D), jnp.float32).at[:, 0].set(1.0)
  onesb = jnp.zeros((_K, _D), jnp.float32).at[:, 64].set(1.0)

  deg = _sc_degrees(src, dst, onesa, onesb, zd)
  g, ns, nd = _tc_prep(x, deg)

  acc = _sc_aggregate(g, src, dst, zd)
  g = _tc_layer(acc, nd, ns, W1, b1.reshape(1, _D), relu=True, scale_next=True)

  acc = _sc_aggregate(g, src, dst, zd)
  g = _tc_layer(acc, nd, ns, W2, b2.reshape(1, _D), relu=True, scale_next=True)

  acc = _sc_aggregate(g, src, dst, zd)
  return _tc_layer(acc, nd, ns, W3, b3.reshape(1, _D), relu=False, scale_next=False)


# speedup vs baseline: 6.5777x; 1.1347x over previous
import functools

import jax
import jax.numpy as jnp
from jax import lax
from jax.experimental import pallas as pl
from jax.experimental.pallas import tpu as pltpu
from jax.experimental.pallas import tpu_sc as plsc

_N = 10000
_E = 320000
_D = 128

_NC = 2
_NS = 16
_NW = _NC * _NS
_EW = _E // _NW
_K = 80
_NCH = _EW // _K
_NP = 10240
_RPS = _NP // _NS
_DG = 16


def _sc_mesh():
  return plsc.VectorSubcoreMesh(core_axis_name="c", subcore_axis_name="s")


@functools.partial(
    pl.kernel,
    out_type=jax.ShapeDtypeStruct((_NC, _NP, _D), jnp.float32),
    mesh=_sc_mesh(),
    scratch_types=[
        pltpu.VMEM((_K,), jnp.int32),
        pltpu.VMEM((_K,), jnp.int32),
        pltpu.VMEM((_K,), jnp.int32),
        pltpu.VMEM((_K,), jnp.int32),
        pltpu.VMEM((_K, _D), jnp.float32),
        pltpu.VMEM((_K, _D), jnp.float32),
        pltpu.VMEM_SHARED((_NP, _D), jnp.float32),
        pltpu.SemaphoreType.DMA,
        pltpu.SemaphoreType.DMA,
    ],
)
def _sc_degrees(src_hbm, dst_hbm, onesa_hbm, onesb_hbm, z_hbm, out_hbm,
                idx_s0, idx_s1, idx_d0, idx_d1, va, vb, acc, sem0, sem1):
  cid = lax.axis_index("c")
  sid = lax.axis_index("s")
  wid = sid * _NC + cid
  row0 = sid * _RPS
  pltpu.sync_copy(z_hbm, acc.at[pl.ds(row0, _RPS)])
  pltpu.sync_copy(onesa_hbm, va)
  pltpu.sync_copy(onesb_hbm, vb)
  plsc.subcore_barrier()

  bufs = ((idx_s0, idx_d0, sem0), (idx_s1, idx_d1, sem1))
  e0 = wid * _EW

  def prefetch(ch, b):
    s, d, sm = bufs[b]
    base = pl.multiple_of(e0 + ch * _K, 8)
    pltpu.async_copy(src_hbm.at[pl.ds(base, _K)], s, sm)
    pltpu.async_copy(dst_hbm.at[pl.ds(base, _K)], d, sm)

  prefetch(0, 0)
  prefetch(1, 1)

  def pair(p, carry):
    for b in range(2):
      ch = p * 2 + b
      s, d, sm = bufs[b]

      @pl.when(ch < _NCH)
      def _():
        pltpu.make_async_copy(src_hbm.at[pl.ds(0, _K)], s, sm).wait()
        pltpu.make_async_copy(dst_hbm.at[pl.ds(0, _K)], d, sm).wait()
        pltpu.sync_copy(va, acc.at[s], add=True)
        pltpu.sync_copy(vb, acc.at[d], add=True)

        @pl.when(ch + 2 < _NCH)
        def _():
          prefetch(ch + 2, b)

    return carry

  lax.fori_loop(0, (_NCH + 1) // 2, pair, 0)
  plsc.subcore_barrier()
  pltpu.sync_copy(acc.at[pl.ds(row0, _RPS)], out_hbm.at[cid, pl.ds(row0, _RPS)])


@functools.partial(
    pl.kernel,
    out_type=jax.ShapeDtypeStruct((_NC, _NP, _D), jnp.float32),
    mesh=_sc_mesh(),
    scratch_types=[
        pltpu.VMEM((_K,), jnp.int32),
        pltpu.VMEM((_K,), jnp.int32),
        pltpu.VMEM((_K,), jnp.int32),
        pltpu.VMEM((_K,), jnp.int32),
        pltpu.VMEM((_K, _D), jnp.float32),
        pltpu.VMEM((_K, _D), jnp.float32),
        pltpu.VMEM_SHARED((_NP, _D), jnp.float32),
        pltpu.SemaphoreType.DMA,
        pltpu.SemaphoreType.DMA,
    ],
)
def _sc_aggregate(g_hbm, src_hbm, dst_hbm, z_hbm, out_hbm,
                  idx_s0, idx_s1, idx_d0, idx_d1, rows0, rows1, acc,
                  sem0, sem1):
  cid = lax.axis_index("c")
  sid = lax.axis_index("s")
  wid = sid * _NC + cid
  row0 = sid * _RPS
  pltpu.sync_copy(z_hbm, acc.at[pl.ds(row0, _RPS)])
  plsc.subcore_barrier()

  bufs = ((idx_s0, idx_d0, rows0, sem0), (idx_s1, idx_d1, rows1, sem1))
  e0 = wid * _EW

  def prefetch(ch, b):
    s, d, r, sm = bufs[b]
    base = pl.multiple_of(e0 + ch * _K, 8)
    pltpu.sync_copy(src_hbm.at[pl.ds(base, _K)], s)
    pltpu.sync_copy(dst_hbm.at[pl.ds(base, _K)], d)
    pltpu.async_copy(g_hbm.at[s], r, sm)

  prefetch(0, 0)
  prefetch(1, 1)

  def pair(p, carry):
    for b in range(2):
      ch = p * 2 + b
      s, d, r, sm = bufs[b]

      @pl.when(ch < _NCH)
      def _():
        pltpu.make_async_copy(g_hbm.at[s], r, sm).wait()
        pltpu.sync_copy(r, acc.at[d], add=True)

        @pl.when(ch + 2 < _NCH)
        def _():
          prefetch(ch + 2, b)

    return carry

  lax.fori_loop(0, (_NCH + 1) // 2, pair, 0)
  plsc.subcore_barrier()
  pltpu.sync_copy(acc.at[pl.ds(row0, _RPS)], out_hbm.at[cid, pl.ds(row0, _RPS)])


_R = 1280
_GB = _NP // _R


def _prep_body(x_ref, deg_ref, g_ref, ns_ref, nd_ref):
  d = deg_ref[0] + deg_ref[1]
  ns = lax.rsqrt(jnp.maximum(d[:, 0:1], 1.0))
  nd = lax.rsqrt(jnp.maximum(d[:, 64:65], 1.0))
  ns_ref[...] = jnp.broadcast_to(ns, (_R, _DG))
  nd_ref[...] = jnp.broadcast_to(nd, (_R, _DG))
  g_ref[...] = x_ref[...] * ns


def _tc_prep(x, deg):
  return pl.pallas_call(
      _prep_body,
      grid=(_GB,),
      in_specs=[
          pl.BlockSpec((_R, _D), lambda i: (i, 0)),
          pl.BlockSpec((_NC, _R, _D), lambda i: (0, i, 0)),
      ],
      out_specs=[
          pl.BlockSpec((_R, _D), lambda i: (i, 0)),
          pl.BlockSpec((_R, _DG), lambda i: (i, 0)),
          pl.BlockSpec((_R, _DG), lambda i: (i, 0)),
      ],
      out_shape=[
          jax.ShapeDtypeStruct((_N, _D), jnp.float32),
          jax.ShapeDtypeStruct((_N, _DG), jnp.float32),
          jax.ShapeDtypeStruct((_N, _DG), jnp.float32),
      ],
  )(x, deg)


def _layer_body(acc_ref, nd_ref, ns_ref, w_ref, b_ref, o_ref, *, relu, scale_next):
  t = (acc_ref[0] + acc_ref[1]) * nd_ref[...][:, :1]
  y = jnp.dot(t, w_ref[...], preferred_element_type=jnp.float32) + b_ref[...]
  if relu:
    y = jnp.maximum(y, 0.0)
  if scale_next:
    y = y * ns_ref[...][:, :1]
  o_ref[...] = y


def _tc_layer(acc, nd, ns, w, b, relu, scale_next):
  body = functools.partial(_layer_body, relu=relu, scale_next=scale_next)
  return pl.pallas_call(
      body,
      grid=(_GB,),
      in_specs=[
          pl.BlockSpec((_NC, _R, _D), lambda i: (0, i, 0)),
          pl.BlockSpec((_R, _DG), lambda i: (i, 0)),
          pl.BlockSpec((_R, _DG), lambda i: (i, 0)),
          pl.BlockSpec((_D, _D), lambda i: (0, 0)),
          pl.BlockSpec((1, _D), lambda i: (0, 0)),
      ],
      out_specs=pl.BlockSpec((_R, _D), lambda i: (i, 0)),
      out_shape=jax.ShapeDtypeStruct((_N, _D), jnp.float32),
  )(acc, nd, ns, w, b)


def kernel(x, edge_index, W1, b1, W2, b2, W3, b3):
  src = edge_index[0].astype(jnp.int32)
  dst = edge_index[1].astype(jnp.int32)
  zd = jnp.zeros((_RPS, _D), jnp.float32)
  onesa = jnp.zeros((_K, _D), jnp.float32).at[:, 0].set(1.0)
  onesb = jnp.zeros((_K, _D), jnp.float32).at[:, 64].set(1.0)

  deg = _sc_degrees(src, dst, onesa, onesb, zd)
  g, ns, nd = _tc_prep(x, deg)

  acc = _sc_aggregate(g, src, dst, zd)
  g = _tc_layer(acc, nd, ns, W1, b1.reshape(1, _D), relu=True, scale_next=True)

  acc = _sc_aggregate(g, src, dst, zd)
  g = _tc_layer(acc, nd, ns, W2, b2.reshape(1, _D), relu=True, scale_next=True)

  acc = _sc_aggregate(g, src, dst, zd)
  return _tc_layer(acc, nd, ns, W3, b3.reshape(1, _D), relu=False, scale_next=False)


# speedup vs baseline: 7.5963x; 1.1549x over previous
import functools

import jax
import jax.numpy as jnp
from jax import lax
from jax.experimental import pallas as pl
from jax.experimental.pallas import tpu as pltpu
from jax.experimental.pallas import tpu_sc as plsc

_N = 10000
_E = 320000
_D = 128

_NC = 2
_NS = 16
_NW = _NC * _NS
_EW = _E // _NW
_K = 80
_NCH = _EW // _K
_NP = 10240
_RPS = _NP // _NS
_DG = 16
_KC = 128
_TCH = _E // _KC
_JMAX = (_TCH + _NW - 1) // _NW


def _sc_mesh():
  return plsc.VectorSubcoreMesh(core_axis_name="c", subcore_axis_name="s")


@functools.partial(
    pl.kernel,
    out_type=jax.ShapeDtypeStruct((_NC, _NP, _D), jnp.float32),
    mesh=_sc_mesh(),
    scratch_types=[
        pltpu.VMEM((_K,), jnp.int32),
        pltpu.VMEM((_K,), jnp.int32),
        pltpu.VMEM((_K,), jnp.int32),
        pltpu.VMEM((_K,), jnp.int32),
        pltpu.VMEM((_K, _D), jnp.float32),
        pltpu.VMEM((_K, _D), jnp.float32),
        pltpu.VMEM_SHARED((_NP, _D), jnp.float32),
        pltpu.SemaphoreType.DMA,
        pltpu.SemaphoreType.DMA,
    ],
)
def _sc_degrees(src_hbm, dst_hbm, onesa_hbm, onesb_hbm, z_hbm, out_hbm,
                idx_s0, idx_s1, idx_d0, idx_d1, va, vb, acc, sem0, sem1):
  cid = lax.axis_index("c")
  sid = lax.axis_index("s")
  wid = sid * _NC + cid
  row0 = sid * _RPS
  pltpu.sync_copy(z_hbm, acc.at[pl.ds(row0, _RPS)])
  pltpu.sync_copy(onesa_hbm, va)
  pltpu.sync_copy(onesb_hbm, vb)
  plsc.subcore_barrier()

  bufs = ((idx_s0, idx_d0, sem0), (idx_s1, idx_d1, sem1))
  e0 = wid * _EW

  def prefetch(ch, b):
    s, d, sm = bufs[b]
    base = pl.multiple_of(e0 + ch * _K, 8)
    pltpu.async_copy(src_hbm.at[pl.ds(base, _K)], s, sm)
    pltpu.async_copy(dst_hbm.at[pl.ds(base, _K)], d, sm)

  prefetch(0, 0)
  prefetch(1, 1)

  def pair(p, carry):
    for b in range(2):
      ch = p * 2 + b
      s, d, sm = bufs[b]

      @pl.when(ch < _NCH)
      def _():
        pltpu.make_async_copy(src_hbm.at[pl.ds(0, _K)], s, sm).wait()
        pltpu.make_async_copy(dst_hbm.at[pl.ds(0, _K)], d, sm).wait()
        pltpu.sync_copy(va, acc.at[s], add=True)
        pltpu.sync_copy(vb, acc.at[d], add=True)

        @pl.when(ch + 2 < _NCH)
        def _():
          prefetch(ch + 2, b)

    return carry

  lax.fori_loop(0, (_NCH + 1) // 2, pair, 0)
  plsc.subcore_barrier()
  pltpu.sync_copy(acc.at[pl.ds(row0, _RPS)], out_hbm.at[cid, pl.ds(row0, _RPS)])


@functools.partial(
    pl.kernel,
    out_type=jax.ShapeDtypeStruct((_NC, _NP, _D), jnp.float32),
    mesh=_sc_mesh(),
    scratch_types=[
        pltpu.VMEM((_KC,), jnp.int32),
        pltpu.VMEM((_KC,), jnp.int32),
        pltpu.VMEM((_KC,), jnp.int32),
        pltpu.VMEM((_KC,), jnp.int32),
        pltpu.VMEM((_KC, _D), jnp.float32),
        pltpu.VMEM((_KC, _D), jnp.float32),
        pltpu.VMEM_SHARED((_NP, _D), jnp.float32),
        pltpu.SemaphoreType.DMA,
        pltpu.SemaphoreType.DMA,
    ],
)
def _sc_aggregate(g_hbm, src_hbm, dst_hbm, z_hbm, out_hbm,
                  idx_s0, idx_s1, idx_d0, idx_d1, rows0, rows1, acc,
                  sem0, sem1):
  cid = lax.axis_index("c")
  sid = lax.axis_index("s")
  wid = sid * _NC + cid
  row0 = sid * _RPS
  pltpu.sync_copy(z_hbm, acc.at[pl.ds(row0, _RPS)])
  plsc.subcore_barrier()

  bufs = ((idx_s0, idx_d0, rows0, sem0), (idx_s1, idx_d1, rows1, sem1))

  def prefetch(j, b):
    s, d, r, sm = bufs[b]
    base = pl.multiple_of((wid + j * _NW) * _KC, 8)
    pltpu.sync_copy(src_hbm.at[pl.ds(base, _KC)], s)
    pltpu.sync_copy(dst_hbm.at[pl.ds(base, _KC)], d)
    pltpu.async_copy(g_hbm.at[s], r, sm)

  prefetch(0, 0)
  prefetch(1, 1)

  def pair(p, carry):
    for b in range(2):
      j = p * 2 + b
      s, d, r, sm = bufs[b]

      @pl.when(wid + j * _NW < _TCH)
      def _():
        pltpu.make_async_copy(g_hbm.at[s], r, sm).wait()
        pltpu.sync_copy(r, acc.at[d], add=True)

        @pl.when(wid + (j + 2) * _NW < _TCH)
        def _():
          prefetch(j + 2, b)

    return carry

  lax.fori_loop(0, (_JMAX + 1) // 2, pair, 0)
  plsc.subcore_barrier()
  pltpu.sync_copy(acc.at[pl.ds(row0, _RPS)], out_hbm.at[cid, pl.ds(row0, _RPS)])


_R = 1280
_GB = _NP // _R


def _prep_body(x_ref, deg_ref, g_ref, ns_ref, nd_ref):
  d = deg_ref[0] + deg_ref[1]
  ns = lax.rsqrt(jnp.maximum(d[:, 0:1], 1.0))
  nd = lax.rsqrt(jnp.maximum(d[:, 64:65], 1.0))
  ns_ref[...] = jnp.broadcast_to(ns, (_R, _DG))
  nd_ref[...] = jnp.broadcast_to(nd, (_R, _DG))
  g_ref[...] = x_ref[...] * ns


def _tc_prep(x, deg):
  return pl.pallas_call(
      _prep_body,
      grid=(_GB,),
      in_specs=[
          pl.BlockSpec((_R, _D), lambda i: (i, 0)),
          pl.BlockSpec((_NC, _R, _D), lambda i: (0, i, 0)),
      ],
      out_specs=[
          pl.BlockSpec((_R, _D), lambda i: (i, 0)),
          pl.BlockSpec((_R, _DG), lambda i: (i, 0)),
          pl.BlockSpec((_R, _DG), lambda i: (i, 0)),
      ],
      out_shape=[
          jax.ShapeDtypeStruct((_N, _D), jnp.float32),
          jax.ShapeDtypeStruct((_N, _DG), jnp.float32),
          jax.ShapeDtypeStruct((_N, _DG), jnp.float32),
      ],
  )(x, deg)


def _layer_body(acc_ref, nd_ref, ns_ref, w_ref, b_ref, o_ref, *, relu, scale_next):
  t = (acc_ref[0] + acc_ref[1]) * nd_ref[...][:, :1]
  y = jnp.dot(t, w_ref[...], preferred_element_type=jnp.float32) + b_ref[...]
  if relu:
    y = jnp.maximum(y, 0.0)
  if scale_next:
    y = y * ns_ref[...][:, :1]
  o_ref[...] = y


def _tc_layer(acc, nd, ns, w, b, relu, scale_next):
  body = functools.partial(_layer_body, relu=relu, scale_next=scale_next)
  return pl.pallas_call(
      body,
      grid=(_GB,),
      in_specs=[
          pl.BlockSpec((_NC, _R, _D), lambda i: (0, i, 0)),
          pl.BlockSpec((_R, _DG), lambda i: (i, 0)),
          pl.BlockSpec((_R, _DG), lambda i: (i, 0)),
          pl.BlockSpec((_D, _D), lambda i: (0, 0)),
          pl.BlockSpec((1, _D), lambda i: (0, 0)),
      ],
      out_specs=pl.BlockSpec((_R, _D), lambda i: (i, 0)),
      out_shape=jax.ShapeDtypeStruct((_N, _D), jnp.float32),
  )(acc, nd, ns, w, b)


def kernel(x, edge_index, W1, b1, W2, b2, W3, b3):
  src = edge_index[0].astype(jnp.int32)
  dst = edge_index[1].astype(jnp.int32)
  zd = jnp.zeros((_RPS, _D), jnp.float32)
  onesa = jnp.zeros((_K, _D), jnp.float32).at[:, 0].set(1.0)
  onesb = jnp.zeros((_K, _D), jnp.float32).at[:, 64].set(1.0)

  deg = _sc_degrees(src, dst, onesa, onesb, zd)
  g, ns, nd = _tc_prep(x, deg)

  acc = _sc_aggregate(g, src, dst, zd)
  g = _tc_layer(acc, nd, ns, W1, b1.reshape(1, _D), relu=True, scale_next=True)

  acc = _sc_aggregate(g, src, dst, zd)
  g = _tc_layer(acc, nd, ns, W2, b2.reshape(1, _D), relu=True, scale_next=True)

  acc = _sc_aggregate(g, src, dst, zd)
  return _tc_layer(acc, nd, ns, W3, b3.reshape(1, _D), relu=False, scale_next=False)


# speedup vs baseline: 7.6403x; 1.0058x over previous
import functools

import jax
import jax.numpy as jnp
from jax import lax
from jax.experimental import pallas as pl
from jax.experimental.pallas import tpu as pltpu
from jax.experimental.pallas import tpu_sc as plsc

_N = 10000
_E = 320000
_D = 128

_NC = 2
_NS = 16
_NW = _NC * _NS
_EW = _E // _NW
_K = 80
_NCH = _EW // _K
_NP = 10240
_RPS = _NP // _NS
_DG = 16
_KC = 80
_TCH = _E // _KC
_JMAX = (_TCH + _NW - 1) // _NW


def _sc_mesh():
  return plsc.VectorSubcoreMesh(core_axis_name="c", subcore_axis_name="s")


@functools.partial(
    pl.kernel,
    out_type=jax.ShapeDtypeStruct((_NC, _NP, _D), jnp.float32),
    mesh=_sc_mesh(),
    scratch_types=[
        pltpu.VMEM((_K,), jnp.int32),
        pltpu.VMEM((_K,), jnp.int32),
        pltpu.VMEM((_K,), jnp.int32),
        pltpu.VMEM((_K,), jnp.int32),
        pltpu.VMEM((_K, _D), jnp.float32),
        pltpu.VMEM((_K, _D), jnp.float32),
        pltpu.VMEM_SHARED((_NP, _D), jnp.float32),
        pltpu.SemaphoreType.DMA,
        pltpu.SemaphoreType.DMA,
    ],
)
def _sc_degrees(src_hbm, dst_hbm, onesa_hbm, onesb_hbm, z_hbm, out_hbm,
                idx_s0, idx_s1, idx_d0, idx_d1, va, vb, acc, sem0, sem1):
  cid = lax.axis_index("c")
  sid = lax.axis_index("s")
  wid = sid * _NC + cid
  row0 = sid * _RPS
  pltpu.sync_copy(z_hbm, acc.at[pl.ds(row0, _RPS)])
  pltpu.sync_copy(onesa_hbm, va)
  pltpu.sync_copy(onesb_hbm, vb)
  plsc.subcore_barrier()

  bufs = ((idx_s0, idx_d0, sem0), (idx_s1, idx_d1, sem1))
  e0 = wid * _EW

  def prefetch(ch, b):
    s, d, sm = bufs[b]
    base = pl.multiple_of(e0 + ch * _K, 8)
    pltpu.async_copy(src_hbm.at[pl.ds(base, _K)], s, sm)
    pltpu.async_copy(dst_hbm.at[pl.ds(base, _K)], d, sm)

  prefetch(0, 0)
  prefetch(1, 1)

  def pair(p, carry):
    for b in range(2):
      ch = p * 2 + b
      s, d, sm = bufs[b]

      @pl.when(ch < _NCH)
      def _():
        pltpu.make_async_copy(src_hbm.at[pl.ds(0, _K)], s, sm).wait()
        pltpu.make_async_copy(dst_hbm.at[pl.ds(0, _K)], d, sm).wait()
        pltpu.sync_copy(va, acc.at[s], add=True)
        pltpu.sync_copy(vb, acc.at[d], add=True)

        @pl.when(ch + 2 < _NCH)
        def _():
          prefetch(ch + 2, b)

    return carry

  lax.fori_loop(0, (_NCH + 1) // 2, pair, 0)
  plsc.subcore_barrier()
  pltpu.sync_copy(acc.at[pl.ds(row0, _RPS)], out_hbm.at[cid, pl.ds(row0, _RPS)])


@functools.partial(
    pl.kernel,
    out_type=jax.ShapeDtypeStruct((_NC, _NP, _D), jnp.float32),
    mesh=_sc_mesh(),
    scratch_types=[
        pltpu.VMEM((_KC,), jnp.int32),
        pltpu.VMEM((_KC,), jnp.int32),
        pltpu.VMEM((_KC,), jnp.int32),
        pltpu.VMEM((_KC,), jnp.int32),
        pltpu.VMEM((_KC,), jnp.int32),
        pltpu.VMEM((_KC,), jnp.int32),
        pltpu.VMEM((_KC,), jnp.int32),
        pltpu.VMEM((_KC,), jnp.int32),
        pltpu.VMEM((4, _KC, _D), jnp.float32),
        pltpu.VMEM_SHARED((_NP, _D), jnp.float32),
        pltpu.SemaphoreType.DMA,
        pltpu.SemaphoreType.DMA,
        pltpu.SemaphoreType.DMA,
        pltpu.SemaphoreType.DMA,
        pltpu.SemaphoreType.DMA,
        pltpu.SemaphoreType.DMA,
        pltpu.SemaphoreType.DMA,
        pltpu.SemaphoreType.DMA,
    ],
)
def _sc_aggregate(g_hbm, src_hbm, dst_hbm, z_hbm, out_hbm,
                  is0, is1, is2, is3, id0, id1, id2, id3, rows, acc,
                  g0, g1, g2, g3, s0, s1, s2, s3):
  cid = lax.axis_index("c")
  sid = lax.axis_index("s")
  wid = sid * _NC + cid
  row0 = sid * _RPS
  pltpu.sync_copy(z_hbm, acc.at[pl.ds(row0, _RPS)])
  plsc.subcore_barrier()

  gsems = (g0, g1, g2, g3)
  ssems = (s0, s1, s2, s3)
  iss = (is0, is1, is2, is3)
  ids = (id0, id1, id2, id3)

  def valid(j):
    return wid + j * _NW < _TCH

  def prefetch(j, b):
    base = pl.multiple_of((wid + j * _NW) * _KC, 8)
    pltpu.sync_copy(src_hbm.at[pl.ds(base, _KC)], iss[b])
    pltpu.sync_copy(dst_hbm.at[pl.ds(base, _KC)], ids[b])
    pltpu.async_copy(g_hbm.at[iss[b]], rows.at[b], gsems[b])

  prefetch(0, 0)
  prefetch(1, 1)

  def visit(j, b, c):
    @pl.when(valid(j))
    def _():
      pltpu.make_async_copy(g_hbm.at[iss[b]], rows.at[b], gsems[b]).wait()
      pltpu.async_copy(rows.at[b], acc.at[ids[b]], ssems[b], add=True)

    @pl.when(valid(j + 2))
    def _():
      @pl.when(j >= 2)
      def _():
        pltpu.make_async_copy(
            rows.at[c], acc.at[ids[c]], ssems[c]).wait()

      prefetch(j + 2, c)

  def quad(q, carry):
    for b in range(4):
      j = q * 4 + b
      visit(j, b, (b + 2) % 4)
    return carry

  lax.fori_loop(0, (_JMAX + 3) // 4, quad, 0)
  for b in range(4):
    pltpu.make_async_copy(rows.at[b], acc.at[ids[b]], ssems[b]).wait()

  plsc.subcore_barrier()
  pltpu.sync_copy(acc.at[pl.ds(row0, _RPS)], out_hbm.at[cid, pl.ds(row0, _RPS)])


_R = 1280
_GB = _NP // _R


def _prep_body(x_ref, deg_ref, g_ref, ns_ref, nd_ref):
  d = deg_ref[0] + deg_ref[1]
  ns = lax.rsqrt(jnp.maximum(d[:, 0:1], 1.0))
  nd = lax.rsqrt(jnp.maximum(d[:, 64:65], 1.0))
  ns_ref[...] = jnp.broadcast_to(ns, (_R, _DG))
  nd_ref[...] = jnp.broadcast_to(nd, (_R, _DG))
  g_ref[...] = x_ref[...] * ns


def _tc_prep(x, deg):
  return pl.pallas_call(
      _prep_body,
      grid=(_GB,),
      in_specs=[
          pl.BlockSpec((_R, _D), lambda i: (i, 0)),
          pl.BlockSpec((_NC, _R, _D), lambda i: (0, i, 0)),
      ],
      out_specs=[
          pl.BlockSpec((_R, _D), lambda i: (i, 0)),
          pl.BlockSpec((_R, _DG), lambda i: (i, 0)),
          pl.BlockSpec((_R, _DG), lambda i: (i, 0)),
      ],
      out_shape=[
          jax.ShapeDtypeStruct((_N, _D), jnp.float32),
          jax.ShapeDtypeStruct((_N, _DG), jnp.float32),
          jax.ShapeDtypeStruct((_N, _DG), jnp.float32),
      ],
  )(x, deg)


def _layer_body(acc_ref, nd_ref, ns_ref, w_ref, b_ref, o_ref, *, relu, scale_next):
  t = (acc_ref[0] + acc_ref[1]) * nd_ref[...][:, :1]
  y = jnp.dot(t, w_ref[...], preferred_element_type=jnp.float32) + b_ref[...]
  if relu:
    y = jnp.maximum(y, 0.0)
  if scale_next:
    y = y * ns_ref[...][:, :1]
  o_ref[...] = y


def _tc_layer(acc, nd, ns, w, b, relu, scale_next):
  body = functools.partial(_layer_body, relu=relu, scale_next=scale_next)
  return pl.pallas_call(
      body,
      grid=(_GB,),
      in_specs=[
          pl.BlockSpec((_NC, _R, _D), lambda i: (0, i, 0)),
          pl.BlockSpec((_R, _DG), lambda i: (i, 0)),
          pl.BlockSpec((_R, _DG), lambda i: (i, 0)),
          pl.BlockSpec((_D, _D), lambda i: (0, 0)),
          pl.BlockSpec((1, _D), lambda i: (0, 0)),
      ],
      out_specs=pl.BlockSpec((_R, _D), lambda i: (i, 0)),
      out_shape=jax.ShapeDtypeStruct((_N, _D), jnp.float32),
  )(acc, nd, ns, w, b)


def kernel(x, edge_index, W1, b1, W2, b2, W3, b3):
  src = edge_index[0].astype(jnp.int32)
  dst = edge_index[1].astype(jnp.int32)
  zd = jnp.zeros((_RPS, _D), jnp.float32)
  onesa = jnp.zeros((_K, _D), jnp.float32).at[:, 0].set(1.0)
  onesb = jnp.zeros((_K, _D), jnp.float32).at[:, 64].set(1.0)

  deg = _sc_degrees(src, dst, onesa, onesb, zd)
  g, ns, nd = _tc_prep(x, deg)

  acc = _sc_aggregate(g, src, dst, zd)
  g = _tc_layer(acc, nd, ns, W1, b1.reshape(1, _D), relu=True, scale_next=True)

  acc = _sc_aggregate(g, src, dst, zd)
  g = _tc_layer(acc, nd, ns, W2, b2.reshape(1, _D), relu=True, scale_next=True)

  acc = _sc_aggregate(g, src, dst, zd)
  return _tc_layer(acc, nd, ns, W3, b3.reshape(1, _D), relu=False, scale_next=False)


# speedup vs baseline: 7.6845x; 1.0058x over previous
import functools

import jax
import jax.numpy as jnp
from jax import lax
from jax.experimental import pallas as pl
from jax.experimental.pallas import tpu as pltpu
from jax.experimental.pallas import tpu_sc as plsc

_N = 10000
_E = 320000
_D = 128

_NC = 2
_NS = 16
_NW = _NC * _NS
_EW = _E // _NW
_K = 80
_NCH = _EW // _K
_NP = 10240
_RPS = _NP // _NS
_DG = 16
_KC = 80
_TCH = _E // _KC
_JMAX = (_TCH + _NW - 1) // _NW


def _sc_mesh():
  return plsc.VectorSubcoreMesh(core_axis_name="c", subcore_axis_name="s")


@functools.partial(
    pl.kernel,
    out_type=jax.ShapeDtypeStruct((_NC, _NP, _D), jnp.float32),
    mesh=_sc_mesh(),
    scratch_types=[
        pltpu.VMEM((_K,), jnp.int32),
        pltpu.VMEM((_K,), jnp.int32),
        pltpu.VMEM((_K,), jnp.int32),
        pltpu.VMEM((_K,), jnp.int32),
        pltpu.VMEM((_K,), jnp.int32),
        pltpu.VMEM((_K,), jnp.int32),
        pltpu.VMEM((_K,), jnp.int32),
        pltpu.VMEM((_K,), jnp.int32),
        pltpu.VMEM((_K, _D), jnp.float32),
        pltpu.VMEM((_K, _D), jnp.float32),
        pltpu.VMEM_SHARED((_NP, _D), jnp.float32),
        pltpu.SemaphoreType.DMA,
        pltpu.SemaphoreType.DMA,
        pltpu.SemaphoreType.DMA,
        pltpu.SemaphoreType.DMA,
        pltpu.SemaphoreType.DMA,
        pltpu.SemaphoreType.DMA,
        pltpu.SemaphoreType.DMA,
        pltpu.SemaphoreType.DMA,
    ],
)
def _sc_degrees(src_hbm, dst_hbm, onesa_hbm, onesb_hbm, z_hbm, out_hbm,
                is0, is1, is2, is3, id0, id1, id2, id3, va, vb, acc,
                i0, i1, i2, i3, s0, s1, s2, s3):
  cid = lax.axis_index("c")
  sid = lax.axis_index("s")
  wid = sid * _NC + cid
  row0 = sid * _RPS
  pltpu.sync_copy(z_hbm, acc.at[pl.ds(row0, _RPS)])
  pltpu.sync_copy(onesa_hbm, va)
  pltpu.sync_copy(onesb_hbm, vb)
  plsc.subcore_barrier()

  iss = (is0, is1, is2, is3)
  ids = (id0, id1, id2, id3)
  isems = (i0, i1, i2, i3)
  ssems = (s0, s1, s2, s3)
  e0 = wid * _EW

  def prefetch(ch, b):
    base = pl.multiple_of(e0 + ch * _K, 8)
    pltpu.async_copy(src_hbm.at[pl.ds(base, _K)], iss[b], isems[b])
    pltpu.async_copy(dst_hbm.at[pl.ds(base, _K)], ids[b], isems[b])

  def drain(b):
    pltpu.make_async_copy(va, acc.at[iss[b]], ssems[b]).wait()
    pltpu.make_async_copy(vb, acc.at[ids[b]], ssems[b]).wait()

  prefetch(0, 0)
  prefetch(1, 1)

  def visit(ch, b, c):
    @pl.when(ch < _NCH)
    def _():
      pltpu.make_async_copy(src_hbm.at[pl.ds(0, _K)], iss[b], isems[b]).wait()
      pltpu.make_async_copy(dst_hbm.at[pl.ds(0, _K)], ids[b], isems[b]).wait()
      pltpu.async_copy(va, acc.at[iss[b]], ssems[b], add=True)
      pltpu.async_copy(vb, acc.at[ids[b]], ssems[b], add=True)

    @pl.when(ch + 2 < _NCH)
    def _():
      @pl.when(ch >= 2)
      def _():
        drain(c)

      prefetch(ch + 2, c)

  def quad(q, carry):
    for b in range(4):
      ch = q * 4 + b
      visit(ch, b, (b + 2) % 4)
    return carry

  lax.fori_loop(0, (_NCH + 3) // 4, quad, 0)
  for b in range(4):
    drain(b)

  plsc.subcore_barrier()
  pltpu.sync_copy(acc.at[pl.ds(row0, _RPS)], out_hbm.at[cid, pl.ds(row0, _RPS)])


@functools.partial(
    pl.kernel,
    out_type=jax.ShapeDtypeStruct((_NC, _NP, _D), jnp.float32),
    mesh=_sc_mesh(),
    scratch_types=[
        pltpu.VMEM((_KC,), jnp.int32),
        pltpu.VMEM((_KC,), jnp.int32),
        pltpu.VMEM((_KC,), jnp.int32),
        pltpu.VMEM((_KC,), jnp.int32),
        pltpu.VMEM((_KC,), jnp.int32),
        pltpu.VMEM((_KC,), jnp.int32),
        pltpu.VMEM((_KC,), jnp.int32),
        pltpu.VMEM((_KC,), jnp.int32),
        pltpu.VMEM((4, _KC, _D), jnp.float32),
        pltpu.VMEM_SHARED((_NP, _D), jnp.float32),
        pltpu.SemaphoreType.DMA,
        pltpu.SemaphoreType.DMA,
        pltpu.SemaphoreType.DMA,
        pltpu.SemaphoreType.DMA,
        pltpu.SemaphoreType.DMA,
        pltpu.SemaphoreType.DMA,
        pltpu.SemaphoreType.DMA,
        pltpu.SemaphoreType.DMA,
    ],
)
def _sc_aggregate(g_hbm, src_hbm, dst_hbm, z_hbm, out_hbm,
                  is0, is1, is2, is3, id0, id1, id2, id3, rows, acc,
                  g0, g1, g2, g3, s0, s1, s2, s3):
  cid = lax.axis_index("c")
  sid = lax.axis_index("s")
  wid = sid * _NC + cid
  row0 = sid * _RPS
  pltpu.sync_copy(z_hbm, acc.at[pl.ds(row0, _RPS)])
  plsc.subcore_barrier()

  gsems = (g0, g1, g2, g3)
  ssems = (s0, s1, s2, s3)
  iss = (is0, is1, is2, is3)
  ids = (id0, id1, id2, id3)

  def valid(j):
    return wid + j * _NW < _TCH

  def prefetch(j, b):
    base = pl.multiple_of((wid + j * _NW) * _KC, 8)
    pltpu.sync_copy(src_hbm.at[pl.ds(base, _KC)], iss[b])
    pltpu.sync_copy(dst_hbm.at[pl.ds(base, _KC)], ids[b])
    pltpu.async_copy(g_hbm.at[iss[b]], rows.at[b], gsems[b])

  prefetch(0, 0)
  prefetch(1, 1)

  def visit(j, b, c):
    @pl.when(valid(j))
    def _():
      pltpu.make_async_copy(g_hbm.at[iss[b]], rows.at[b], gsems[b]).wait()
      pltpu.async_copy(rows.at[b], acc.at[ids[b]], ssems[b], add=True)

    @pl.when(valid(j + 2))
    def _():
      @pl.when(j >= 2)
      def _():
        pltpu.make_async_copy(
            rows.at[c], acc.at[ids[c]], ssems[c]).wait()

      prefetch(j + 2, c)

  def quad(q, carry):
    for b in range(4):
      j = q * 4 + b
      visit(j, b, (b + 2) % 4)
    return carry

  lax.fori_loop(0, (_JMAX + 3) // 4, quad, 0)
  for b in range(4):
    pltpu.make_async_copy(rows.at[b], acc.at[ids[b]], ssems[b]).wait()

  plsc.subcore_barrier()
  pltpu.sync_copy(acc.at[pl.ds(row0, _RPS)], out_hbm.at[cid, pl.ds(row0, _RPS)])


_R = 1280
_GB = _NP // _R


def _prep_body(x_ref, deg_ref, g_ref, ns_ref, nd_ref):
  d = deg_ref[0] + deg_ref[1]
  ns = lax.rsqrt(jnp.maximum(d[:, 0:1], 1.0))
  nd = lax.rsqrt(jnp.maximum(d[:, 64:65], 1.0))
  ns_ref[...] = jnp.broadcast_to(ns, (_R, _DG))
  nd_ref[...] = jnp.broadcast_to(nd, (_R, _DG))
  g_ref[...] = x_ref[...] * ns


def _tc_prep(x, deg):
  return pl.pallas_call(
      _prep_body,
      grid=(_GB,),
      in_specs=[
          pl.BlockSpec((_R, _D), lambda i: (i, 0)),
          pl.BlockSpec((_NC, _R, _D), lambda i: (0, i, 0)),
      ],
      out_specs=[
          pl.BlockSpec((_R, _D), lambda i: (i, 0)),
          pl.BlockSpec((_R, _DG), lambda i: (i, 0)),
          pl.BlockSpec((_R, _DG), lambda i: (i, 0)),
      ],
      out_shape=[
          jax.ShapeDtypeStruct((_N, _D), jnp.float32),
          jax.ShapeDtypeStruct((_N, _DG), jnp.float32),
          jax.ShapeDtypeStruct((_N, _DG), jnp.float32),
      ],
  )(x, deg)


def _layer_body(acc_ref, nd_ref, ns_ref, w_ref, b_ref, o_ref, *, relu, scale_next):
  t = (acc_ref[0] + acc_ref[1]) * nd_ref[...][:, :1]
  y = jnp.dot(t, w_ref[...], preferred_element_type=jnp.float32) + b_ref[...]
  if relu:
    y = jnp.maximum(y, 0.0)
  if scale_next:
    y = y * ns_ref[...][:, :1]
  o_ref[...] = y


def _tc_layer(acc, nd, ns, w, b, relu, scale_next):
  body = functools.partial(_layer_body, relu=relu, scale_next=scale_next)
  return pl.pallas_call(
      body,
      grid=(_GB,),
      in_specs=[
          pl.BlockSpec((_NC, _R, _D), lambda i: (0, i, 0)),
          pl.BlockSpec((_R, _DG), lambda i: (i, 0)),
          pl.BlockSpec((_R, _DG), lambda i: (i, 0)),
          pl.BlockSpec((_D, _D), lambda i: (0, 0)),
          pl.BlockSpec((1, _D), lambda i: (0, 0)),
      ],
      out_specs=pl.BlockSpec((_R, _D), lambda i: (i, 0)),
      out_shape=jax.ShapeDtypeStruct((_N, _D), jnp.float32),
  )(acc, nd, ns, w, b)


def kernel(x, edge_index, W1, b1, W2, b2, W3, b3):
  src = edge_index[0].astype(jnp.int32)
  dst = edge_index[1].astype(jnp.int32)
  zd = jnp.zeros((_RPS, _D), jnp.float32)
  onesa = jnp.zeros((_K, _D), jnp.float32).at[:, 0].set(1.0)
  onesb = jnp.zeros((_K, _D), jnp.float32).at[:, 64].set(1.0)

  deg = _sc_degrees(src, dst, onesa, onesb, zd)
  g, ns, nd = _tc_prep(x, deg)

  acc = _sc_aggregate(g, src, dst, zd)
  g = _tc_layer(acc, nd, ns, W1, b1.reshape(1, _D), relu=True, scale_next=True)

  acc = _sc_aggregate(g, src, dst, zd)
  g = _tc_layer(acc, nd, ns, W2, b2.reshape(1, _D), relu=True, scale_next=True)

  acc = _sc_aggregate(g, src, dst, zd)
  return _tc_layer(acc, nd, ns, W3, b3.reshape(1, _D), relu=False, scale_next=False)
